# baseline (speedup 1.0000x reference)
import functools

import jax
import jax.numpy as jnp
from jax import lax
from jax.experimental import pallas as pl
from jax.experimental.pallas import tpu as pltpu
from jax.experimental.pallas import tpu_sc as plsc

B, H, W = 16, 64, 64
NCELL = 4
EMB = 16
NUM_LAYERS = 2
DIST2 = 900.0
EPS = 1e-06
PIX = H * W
HPIX = PIX // 2
LANES = 16

XA = 0
XB = 4096
X0 = 8192
X1 = 10240
NI32 = 12288
WENC = 0
WSELF = 80
WNBR = 592
WDEC = 1104
STAGE = 1120
PART = 1136
ADJ = 1152
TABLE = 1168
PROBS = 1488
PREDF = PROBS + NCELL * HPIX
NF32 = PREDF + HPIX
SH_M = 0
SH_P = 256
SH_T = 512
NSH = 512 + 16 * 160


def _sc_body(x_hbm, xt_hbm, wcat_hbm, out_hbm,
             vi, vf, sh, sem):
    core = lax.axis_index("c")
    sub = lax.axis_index("s")
    b = core * 8 + (sub >> 1)
    half = sub & 1
    partner = sub ^ 1

    iota = lax.iota(jnp.int32, LANES)
    iotaf = iota.astype(jnp.float32)
    zf = jnp.zeros((LANES,), jnp.float32)

    def _shuf(v, idx):
        return lax.gather(
            v, idx[:, None],
            dimension_numbers=lax.GatherDimensionNumbers(
                offset_dims=(), collapsed_slice_dims=(0,),
                start_index_map=(0,)),
            slice_sizes=(1,),
            mode=lax.GatherScatterMode.PROMISE_IN_BOUNDS)

    def _bmax(v):
        for s in (8, 4, 2, 1):
            v = jnp.maximum(v, _shuf(v, iota ^ s))
        return v

    def _bsum(v):
        for s in (8, 4, 2, 1):
            v = v + _shuf(v, iota ^ s)
        return v

    hA = pltpu.async_copy(x_hbm.at[pl.ds(2 * sub * PIX, PIX)],
                          vi.at[pl.ds(XA, PIX)], sem)
    hB = pltpu.async_copy(xt_hbm.at[pl.ds(2 * sub * PIX, PIX)],
                          vi.at[pl.ds(XB, PIX)], sem)
    x_off = 2 * b * PIX + half * HPIX
    hX = pltpu.async_copy(x_hbm.at[pl.ds(x_off, HPIX)],
                          vi.at[pl.ds(X0, HPIX)], sem)
    hY = pltpu.async_copy(x_hbm.at[pl.ds(PIX + x_off, HPIX)],
                          vi.at[pl.ds(X1, HPIX)], sem)
    hW = pltpu.async_copy(wcat_hbm, vf.at[pl.ds(WENC, 1120)], sem)

    hA.wait()
    hB.wait()

    def _mx_step(i, acc):
        a = jnp.maximum(acc, vi[pl.ds(XA + i * LANES, LANES)])
        return jnp.maximum(a, vi[pl.ds(XB + i * LANES, LANES)])

    acc0 = jnp.full((LANES,), -(2 ** 31 - 1), jnp.int32)
    accm = lax.fori_loop(0, PIX // LANES, _mx_step, acc0, unroll=8)
    vf[pl.ds(STAGE, LANES)] = _bmax(accm.astype(jnp.float32))
    pltpu.sync_copy(vf.at[pl.ds(STAGE, LANES)],
                    sh.at[pl.ds(SH_M + sub * LANES, LANES)])
    plsc.subcore_barrier()
    pltpu.sync_copy(sh.at[pl.ds(SH_M, 256)], vf.at[pl.ds(PROBS, 256)])

    def _mx2_step(i, acc):
        return jnp.maximum(acc, vf[pl.ds(PROBS + i * LANES, LANES)])

    accg = lax.fori_loop(0, LANES, _mx2_step,
                         jnp.full((LANES,), -3.4e38, jnp.float32), unroll=4)
    shift = _bmax(accg).astype(jnp.int32) - 3

    hX.wait()
    hY.wait()

    def _red_row(r, carry):
        accs = list(carry)
        rowv = jnp.broadcast_to((half * 32 + r).astype(jnp.float32), (LANES,))
        for jv in range(4):
            cid = vi[pl.ds(X0 + r * 64 + jv * 16, LANES)] + shift
            colv = iotaf + float(jv * 16)
            for c in range(NCELL):
                msk = cid == c
                accs[c] = accs[c] + jnp.where(msk, 1.0, 0.0)
                accs[4 + c] = accs[4 + c] + jnp.where(msk, rowv, zf)
                accs[8 + c] = accs[8 + c] + jnp.where(msk, colv, zf)
        return tuple(accs)

    accs = lax.fori_loop(0, 32, _red_row, tuple(zf for _ in range(12)),
                         unroll=2)
    pv = zf
    for idx in range(12):
        pv = jnp.where(iota == idx, _bsum(accs[idx]), pv)
    vf[pl.ds(STAGE, LANES)] = pv
    pltpu.sync_copy(vf.at[pl.ds(STAGE, LANES)],
                    sh.at[pl.ds(SH_P + sub * LANES, LANES)])
    plsc.subcore_barrier()
    pltpu.sync_copy(sh.at[pl.ds(SH_P + partner * LANES, LANES)],
                    vf.at[pl.ds(PART, LANES)])
    tot = vf[pl.ds(STAGE, LANES)] + vf[pl.ds(PART, LANES)]

    qs = iota >> 2
    qd = iota & 3
    cnt_s = _shuf(tot, qs)
    cnt_d = _shuf(tot, qd)
    ch_s = _shuf(tot, qs + 4) / cnt_s
    ch_d = _shuf(tot, qd + 4) / cnt_d
    cw_s = _shuf(tot, qs + 8) / cnt_s
    cw_d = _shuf(tot, qd + 8) / cnt_d
    dh = ch_s - ch_d
    dw = cw_s - cw_d
    d2 = dh * dh + dw * dw
    cntm = jnp.where(iota < 4, tot, jnp.full((LANES,), -1.0, jnp.float32))
    ism = cntm == _bmax(cntm)
    score = jnp.where(ism, 16 - iota, jnp.zeros((LANES,), jnp.int32))
    med = 16 - _bmax(score)
    ok = ((d2 <= DIST2) & (cnt_s > 0.0) & (cnt_d > 0.0)
          & (qs != med) & (qd != med))
    vf[pl.ds(ADJ, LANES)] = jnp.where(ok, 1.0, 0.0)

    hW.wait()
    base_k = half * 10

    def _mlp_class(k, carry):
        kk = base_k + k
        t = kk & 3
        p = kk >> 2
        wrow_t = vf[pl.ds(WENC + (1 + t) * EMB, EMB)]
        wrow_0 = vf[pl.ds(WENC, EMB)]
        adjv = vf[pl.ds(ADJ, LANES)]
        hs = [jnp.maximum(wrow_t + wrow_0 * jnp.where(p == c, 1.0, 0.0), 0.0)
              for c in range(NCELL)]
        for l in range(NUM_LAYERS):
            aggs = []
            for d in range(NCELL):
                agg = zf
                for s in range(NCELL):
                    agg = agg + adjv[s * 4 + d] * hs[s]
                aggs.append(agg)
            new_hs = []
            for d in range(NCELL):
                acc = zf
                for e in range(EMB):
                    acc = acc + hs[d][e] * vf[pl.ds(WSELF + l * 256 + e * EMB, EMB)]
                    acc = acc + aggs[d][e] * vf[pl.ds(WNBR + l * 256 + e * EMB, EMB)]
                new_hs.append(jnp.maximum(acc, 0.0))
            hs = new_hs
        wd = vf[pl.ds(WDEC, EMB)]
        tv = zf
        for c in range(NCELL):
            tv = jnp.where(iota == c, _bsum(hs[c] * wd), tv)
        vf[pl.ds(TABLE + kk * LANES, LANES)] = tv
        return carry

    lax.fori_loop(0, 10, _mlp_class, 0)
    pltpu.sync_copy(vf.at[pl.ds(TABLE + half * 160, 160)],
                    sh.at[pl.ds(SH_T + sub * 160, 160)])
    plsc.subcore_barrier()
    pltpu.sync_copy(sh.at[pl.ds(SH_T + partner * 160, 160)],
                    vf.at[pl.ds(TABLE + (1 - half) * 160, 160)])

    T0 = [zf, zf, zf, zf]
    T1 = [zf, zf, zf, zf]
    for k in range(16):
        row_k = vf[pl.ds(TABLE + k * LANES, LANES)]
        for c in range(NCELL):
            T0[c] = jnp.where(iota == k, row_k[c], T0[c])
    for k in range(4):
        row_k = vf[pl.ds(TABLE + (16 + k) * LANES, LANES)]
        for c in range(NCELL):
            T1[c] = jnp.where(iota == k, row_k[c], T1[c])
    mxg = _bmax(jnp.maximum(
        jnp.maximum(jnp.maximum(T0[0], T0[1]), jnp.maximum(T0[2], T0[3])),
        jnp.maximum(jnp.maximum(T1[0], T1[1]), jnp.maximum(T1[2], T1[3]))))
    E0 = [jnp.exp(v - mxg) for v in T0]
    E1 = [jnp.exp(v - mxg) for v in T1]

    def _row(r, carry):
        rbase = r * 64
        Ls = []
        for jv in range(4):
            cid = vi[pl.ds(X0 + rbase + jv * 16, LANES)] + shift
            t = vi[pl.ds(X1 + rbase + jv * 16, LANES)]
            valid = cid >= 0
            k0 = jnp.where(valid, cid * 4 + t, 0)
            Ls.append([jnp.where(valid, _shuf(E0[c], k0), _shuf(E1[c], t))
                       for c in range(NCELL)])
        for c in range(NCELL):
            l0, l1, l2, l3 = (Ls[0][c], Ls[1][c], Ls[2][c], Ls[3][c])
            inv = 1.0 / _bsum((l0 + l1) + (l2 + l3))
            for jv in range(4):
                vf[pl.ds(PROBS + c * HPIX + rbase + jv * 16, LANES)] = (
                    Ls[jv][c] * inv + EPS)
        for jv in range(4):
            a0, a1, a2, a3 = (Ls[jv][0], Ls[jv][1], Ls[jv][2], Ls[jv][3])
            pm = jnp.maximum(jnp.maximum(a0, a1), jnp.maximum(a2, a3))
            arg = jnp.full((LANES,), 3, jnp.int32)
            arg = jnp.where(a2 == pm, 2, arg)
            arg = jnp.where(a1 == pm, 1, arg)
            arg = jnp.where(a0 == pm, 0, arg)
            vf[pl.ds(PREDF + rbase + jv * 16, LANES)] = arg.astype(
                jnp.float32)
        return carry

    lax.fori_loop(0, 32, _row, 0, unroll=2)

    hs_out = []
    for c in range(NCELL):
        hs_out.append(pltpu.async_copy(
            vf.at[pl.ds(PROBS + c * HPIX, HPIX)],
            out_hbm.at[pl.ds(((b * NCELL + c) * H + half * 32) * W, HPIX)],
            sem))
    hs_out.append(pltpu.async_copy(
        vf.at[pl.ds(PREDF, HPIX)],
        out_hbm.at[pl.ds(B * NCELL * H * W + (b * H + half * 32) * W, HPIX)],
        sem))
    for h in hs_out:
        h.wait()


@jax.jit
def kernel(x, x_true, W_enc, W_self, W_nbr, W_dec):
    xflat = x.reshape(-1)
    xtflat = x_true.reshape(-1)
    wcat = jnp.concatenate([W_enc.reshape(-1), W_self.reshape(-1),
                            W_nbr.reshape(-1), W_dec.reshape(-1)])

    mesh = plsc.VectorSubcoreMesh(core_axis_name="c", subcore_axis_name="s")
    run = functools.partial(
        pl.kernel,
        mesh=mesh,
        out_type=[
            jax.ShapeDtypeStruct((B * NCELL * H * W + B * H * W,),
                                 jnp.float32),
        ],
        scratch_types=[
            pltpu.VMEM((NI32,), jnp.int32),
            pltpu.VMEM((NF32,), jnp.float32),
            pltpu.VMEM_SHARED((NSH,), jnp.float32),
            pltpu.SemaphoreType.DMA,
        ],
    )(_sc_body)

    (outf,) = run(xflat, xtflat, wcat)
    probs = outf[:B * NCELL * H * W].reshape(B, NCELL, H, W)
    pred = outf[B * NCELL * H * W:].astype(jnp.int32).reshape(B, H, W)
    return probs, pred


# speedup vs baseline: 1.1057x; 1.1057x over previous
import functools

import jax
import jax.numpy as jnp
from jax import lax
from jax.experimental import pallas as pl
from jax.experimental.pallas import tpu as pltpu
from jax.experimental.pallas import tpu_sc as plsc

B, H, W = 16, 64, 64
NCELL = 4
EMB = 16
NUM_LAYERS = 2
DIST2 = 900.0
EPS = 1e-06
PIX = H * W
HPIX = PIX // 2
LANES = 16

XA = 0
XB = 4096
X0 = 8192
X1 = 10240
PRED = 12288
NI32 = 14336
WENC = 0
WSELF = 80
WNBR = 592
WDEC = 1104
STAGE = 1120
PART = 1136
ADJ = 1152
TABLE = 1168
PROBS = 1488
NF32 = PROBS + NCELL * HPIX
SH_M = 0
SH_P = 256
SH_T = 512
NSH = 512 + 16 * 160


def _sc_body(x_hbm, xt_hbm, wcat_hbm, probs_hbm, pred_hbm,
             vi, vf, sh, sem, sem_out):
    core = lax.axis_index("c")
    sub = lax.axis_index("s")
    b = core * 8 + (sub >> 1)
    half = sub & 1
    partner = sub ^ 1

    iota = lax.iota(jnp.int32, LANES)
    iotaf = iota.astype(jnp.float32)
    zf = jnp.zeros((LANES,), jnp.float32)

    def _shuf(v, idx):
        return lax.gather(
            v, idx[:, None],
            dimension_numbers=lax.GatherDimensionNumbers(
                offset_dims=(), collapsed_slice_dims=(0,),
                start_index_map=(0,)),
            slice_sizes=(1,),
            mode=lax.GatherScatterMode.PROMISE_IN_BOUNDS)

    def _bmax(v):
        for s in (8, 4, 2, 1):
            v = jnp.maximum(v, _shuf(v, iota ^ s))
        return v

    def _bsum(v):
        for s in (8, 4, 2, 1):
            v = v + _shuf(v, iota ^ s)
        return v

    hA = pltpu.async_copy(x_hbm.at[pl.ds(2 * sub * PIX, PIX)],
                          vi.at[pl.ds(XA, PIX)], sem)
    hB = pltpu.async_copy(xt_hbm.at[pl.ds(2 * sub * PIX, PIX)],
                          vi.at[pl.ds(XB, PIX)], sem)
    x_off = 2 * b * PIX + half * HPIX
    hX = pltpu.async_copy(x_hbm.at[pl.ds(x_off, HPIX)],
                          vi.at[pl.ds(X0, HPIX)], sem)
    hY = pltpu.async_copy(x_hbm.at[pl.ds(PIX + x_off, HPIX)],
                          vi.at[pl.ds(X1, HPIX)], sem)
    hW = pltpu.async_copy(wcat_hbm, vf.at[pl.ds(WENC, 1120)], sem)

    hA.wait()
    hB.wait()

    def _mx_step(i, acc):
        a = jnp.maximum(acc, vi[pl.ds(XA + i * LANES, LANES)])
        return jnp.maximum(a, vi[pl.ds(XB + i * LANES, LANES)])

    acc0 = jnp.full((LANES,), -(2 ** 31 - 1), jnp.int32)
    accm = lax.fori_loop(0, PIX // LANES, _mx_step, acc0, unroll=8)
    vf[pl.ds(STAGE, LANES)] = _bmax(accm.astype(jnp.float32))
    pltpu.sync_copy(vf.at[pl.ds(STAGE, LANES)],
                    sh.at[pl.ds(SH_M + sub * LANES, LANES)])
    plsc.subcore_barrier()
    pltpu.sync_copy(sh.at[pl.ds(SH_M, 256)], vf.at[pl.ds(PROBS, 256)])

    def _mx2_step(i, acc):
        return jnp.maximum(acc, vf[pl.ds(PROBS + i * LANES, LANES)])

    accg = lax.fori_loop(0, LANES, _mx2_step,
                         jnp.full((LANES,), -3.4e38, jnp.float32), unroll=4)
    shift = _bmax(accg).astype(jnp.int32) - 3

    hX.wait()
    hY.wait()

    def _red_row(r, carry):
        accs = list(carry)
        rowv = jnp.broadcast_to((half * 32 + r).astype(jnp.float32), (LANES,))
        for jv in range(4):
            cid = vi[pl.ds(X0 + r * 64 + jv * 16, LANES)] + shift
            colv = iotaf + float(jv * 16)
            for c in range(NCELL):
                msk = cid == c
                accs[c] = accs[c] + jnp.where(msk, 1.0, 0.0)
                accs[4 + c] = accs[4 + c] + jnp.where(msk, rowv, zf)
                accs[8 + c] = accs[8 + c] + jnp.where(msk, colv, zf)
        return tuple(accs)

    accs = lax.fori_loop(0, 32, _red_row, tuple(zf for _ in range(12)),
                         unroll=2)
    pv = zf
    for idx in range(12):
        pv = jnp.where(iota == idx, _bsum(accs[idx]), pv)
    vf[pl.ds(STAGE, LANES)] = pv
    pltpu.sync_copy(vf.at[pl.ds(STAGE, LANES)],
                    sh.at[pl.ds(SH_P + sub * LANES, LANES)])
    plsc.subcore_barrier()
    pltpu.sync_copy(sh.at[pl.ds(SH_P + partner * LANES, LANES)],
                    vf.at[pl.ds(PART, LANES)])
    tot = vf[pl.ds(STAGE, LANES)] + vf[pl.ds(PART, LANES)]

    qs = iota >> 2
    qd = iota & 3
    cnt_s = _shuf(tot, qs)
    cnt_d = _shuf(tot, qd)
    ch_s = _shuf(tot, qs + 4) / cnt_s
    ch_d = _shuf(tot, qd + 4) / cnt_d
    cw_s = _shuf(tot, qs + 8) / cnt_s
    cw_d = _shuf(tot, qd + 8) / cnt_d
    dh = ch_s - ch_d
    dw = cw_s - cw_d
    d2 = dh * dh + dw * dw
    cntm = jnp.where(iota < 4, tot, jnp.full((LANES,), -1.0, jnp.float32))
    ism = cntm == _bmax(cntm)
    score = jnp.where(ism, 16 - iota, jnp.zeros((LANES,), jnp.int32))
    med = 16 - _bmax(score)
    ok = ((d2 <= DIST2) & (cnt_s > 0.0) & (cnt_d > 0.0)
          & (qs != med) & (qd != med))
    vf[pl.ds(ADJ, LANES)] = jnp.where(ok, 1.0, 0.0)

    hW.wait()
    base_k = half * 10

    def _mlp_class(k, carry):
        kk = base_k + k
        t = kk & 3
        p = kk >> 2
        wrow_t = vf[pl.ds(WENC + (1 + t) * EMB, EMB)]
        wrow_0 = vf[pl.ds(WENC, EMB)]
        adjv = vf[pl.ds(ADJ, LANES)]
        hs = [jnp.maximum(wrow_t + wrow_0 * jnp.where(p == c, 1.0, 0.0), 0.0)
              for c in range(NCELL)]
        for l in range(NUM_LAYERS):
            aggs = []
            for d in range(NCELL):
                agg = zf
                for s in range(NCELL):
                    agg = agg + adjv[s * 4 + d] * hs[s]
                aggs.append(agg)
            new_hs = []
            for d in range(NCELL):
                acc = zf
                for e in range(EMB):
                    acc = acc + hs[d][e] * vf[pl.ds(WSELF + l * 256 + e * EMB, EMB)]
                    acc = acc + aggs[d][e] * vf[pl.ds(WNBR + l * 256 + e * EMB, EMB)]
                new_hs.append(jnp.maximum(acc, 0.0))
            hs = new_hs
        wd = vf[pl.ds(WDEC, EMB)]
        tv = zf
        for c in range(NCELL):
            tv = jnp.where(iota == c, _bsum(hs[c] * wd), tv)
        vf[pl.ds(TABLE + kk * LANES, LANES)] = tv
        return carry

    lax.fori_loop(0, 10, _mlp_class, 0)
    pltpu.sync_copy(vf.at[pl.ds(TABLE + half * 160, 160)],
                    sh.at[pl.ds(SH_T + sub * 160, 160)])
    plsc.subcore_barrier()
    pltpu.sync_copy(sh.at[pl.ds(SH_T + partner * 160, 160)],
                    vf.at[pl.ds(TABLE + (1 - half) * 160, 160)])

    T0 = [zf, zf, zf, zf]
    T1 = [zf, zf, zf, zf]
    for k in range(16):
        row_k = vf[pl.ds(TABLE + k * LANES, LANES)]
        for c in range(NCELL):
            T0[c] = jnp.where(iota == k, row_k[c], T0[c])
    for k in range(4):
        row_k = vf[pl.ds(TABLE + (16 + k) * LANES, LANES)]
        for c in range(NCELL):
            T1[c] = jnp.where(iota == k, row_k[c], T1[c])
    mxg = _bmax(jnp.maximum(
        jnp.maximum(jnp.maximum(T0[0], T0[1]), jnp.maximum(T0[2], T0[3])),
        jnp.maximum(jnp.maximum(T1[0], T1[1]), jnp.maximum(T1[2], T1[3]))))
    E0 = [jnp.exp(v - mxg) for v in T0]
    E1 = [jnp.exp(v - mxg) for v in T1]

    def _row(r, carry):
        rbase = r * 64
        Ls = []
        for jv in range(4):
            cid = vi[pl.ds(X0 + rbase + jv * 16, LANES)] + shift
            t = vi[pl.ds(X1 + rbase + jv * 16, LANES)]
            valid = cid >= 0
            k0 = jnp.where(valid, cid * 4 + t, 0)
            Ls.append([jnp.where(valid, _shuf(E0[c], k0), _shuf(E1[c], t))
                       for c in range(NCELL)])
        for c in range(NCELL):
            l0, l1, l2, l3 = (Ls[0][c], Ls[1][c], Ls[2][c], Ls[3][c])
            inv = 1.0 / _bsum((l0 + l1) + (l2 + l3))
            for jv in range(4):
                vf[pl.ds(PROBS + c * HPIX + rbase + jv * 16, LANES)] = (
                    Ls[jv][c] * inv + EPS)
        for jv in range(4):
            a0, a1, a2, a3 = (Ls[jv][0], Ls[jv][1], Ls[jv][2], Ls[jv][3])
            pm = jnp.maximum(jnp.maximum(a0, a1), jnp.maximum(a2, a3))
            arg = jnp.full((LANES,), 3, jnp.int32)
            arg = jnp.where(a2 == pm, 2, arg)
            arg = jnp.where(a1 == pm, 1, arg)
            arg = jnp.where(a0 == pm, 0, arg)
            vi[pl.ds(PRED + rbase + jv * 16, LANES)] = arg
        return carry

    lax.fori_loop(0, 32, _row, 0, unroll=2)

    hs_out = []
    for c in range(NCELL):
        hs_out.append(pltpu.async_copy(
            vf.at[pl.ds(PROBS + c * HPIX, HPIX)],
            probs_hbm.at[pl.ds(((b * NCELL + c) * H + half * 32) * W, HPIX)],
            sem_out))
    hs_out.append(pltpu.async_copy(
        vi.at[pl.ds(PRED, HPIX)],
        pred_hbm.at[pl.ds((b * H + half * 32) * W, HPIX)], sem_out))
    for h in hs_out:
        h.wait()


@jax.jit
def kernel(x, x_true, W_enc, W_self, W_nbr, W_dec):
    xflat = x.reshape(-1)
    xtflat = x_true.reshape(-1)
    wcat = jnp.concatenate([W_enc.reshape(-1), W_self.reshape(-1),
                            W_nbr.reshape(-1), W_dec.reshape(-1)])

    mesh = plsc.VectorSubcoreMesh(core_axis_name="c", subcore_axis_name="s")
    run = functools.partial(
        pl.kernel,
        mesh=mesh,
        out_type=[
            jax.ShapeDtypeStruct((B * NCELL * H * W,), jnp.float32),
            jax.ShapeDtypeStruct((B * H * W,), jnp.int32),
        ],
        scratch_types=[
            pltpu.VMEM((NI32,), jnp.int32),
            pltpu.VMEM((NF32,), jnp.float32),
            pltpu.VMEM_SHARED((NSH,), jnp.float32),
            pltpu.SemaphoreType.DMA,
            pltpu.SemaphoreType.DMA,
        ],
    )(_sc_body)

    probsf, predf = run(xflat, xtflat, wcat)
    return probsf.reshape(B, NCELL, H, W), predf.reshape(B, H, W)


# speedup vs baseline: 1.1202x; 1.0131x over previous
import functools

import jax
import jax.numpy as jnp
from jax import lax
from jax.experimental import pallas as pl
from jax.experimental.pallas import tpu as pltpu
from jax.experimental.pallas import tpu_sc as plsc

B, H, W = 16, 64, 64
NCELL = 4
EMB = 16
NUM_LAYERS = 2
DIST2 = 900.0
EPS = 1e-06
PIX = H * W
HPIX = PIX // 2
LANES = 16

XA = 0
XB = 4096
X0 = 8192
X1 = 10240
PRED = 12288
NI32 = 14336
WENC = 0
WSELF = 80
WNBR = 592
WDEC = 1104
STAGE = 1120
PART = 1136
ADJ = 1152
TABLE = 1168
PROBS = 1488
NF32 = PROBS + NCELL * HPIX
SH_M = 0
SH_P = 256
SH_T = 512
NSH = 512 + 16 * 160


def _sc_body(x_hbm, xt_hbm, wcat_hbm, probs_hbm, pred_hbm,
             vi, vf, sh, sem, sem_out):
    core = lax.axis_index("c")
    sub = lax.axis_index("s")
    b = core * 8 + (sub >> 1)
    half = sub & 1
    partner = sub ^ 1

    iota = lax.iota(jnp.int32, LANES)
    iotaf = iota.astype(jnp.float32)
    zf = jnp.zeros((LANES,), jnp.float32)

    def _shuf(v, idx):
        return lax.gather(
            v, idx[:, None],
            dimension_numbers=lax.GatherDimensionNumbers(
                offset_dims=(), collapsed_slice_dims=(0,),
                start_index_map=(0,)),
            slice_sizes=(1,),
            mode=lax.GatherScatterMode.PROMISE_IN_BOUNDS)

    def _bmax(v):
        for s in (8, 4, 2, 1):
            v = jnp.maximum(v, _shuf(v, iota ^ s))
        return v

    def _bsum(v):
        for s in (8, 4, 2, 1):
            v = v + _shuf(v, iota ^ s)
        return v

    hA = pltpu.async_copy(x_hbm.at[pl.ds(2 * sub * PIX, PIX)],
                          vi.at[pl.ds(XA, PIX)], sem)
    hB = pltpu.async_copy(xt_hbm.at[pl.ds(2 * sub * PIX, PIX)],
                          vi.at[pl.ds(XB, PIX)], sem)
    x_off = 2 * b * PIX + half * HPIX
    hX = pltpu.async_copy(x_hbm.at[pl.ds(x_off, HPIX)],
                          vi.at[pl.ds(X0, HPIX)], sem)
    hY = pltpu.async_copy(x_hbm.at[pl.ds(PIX + x_off, HPIX)],
                          vi.at[pl.ds(X1, HPIX)], sem)
    hW = pltpu.async_copy(wcat_hbm, vf.at[pl.ds(WENC, 1120)], sem)

    hA.wait()
    hB.wait()

    def _mx_step(i, acc):
        a = jnp.maximum(acc, vi[pl.ds(XA + i * LANES, LANES)])
        return jnp.maximum(a, vi[pl.ds(XB + i * LANES, LANES)])

    acc0 = jnp.full((LANES,), -(2 ** 31 - 1), jnp.int32)
    accm = lax.fori_loop(0, PIX // LANES, _mx_step, acc0, unroll=8)
    vf[pl.ds(STAGE, LANES)] = _bmax(accm.astype(jnp.float32))
    pltpu.sync_copy(vf.at[pl.ds(STAGE, LANES)],
                    sh.at[pl.ds(SH_M + sub * LANES, LANES)])
    plsc.subcore_barrier()
    pltpu.sync_copy(sh.at[pl.ds(SH_M, 256)], vf.at[pl.ds(PROBS, 256)])

    def _mx2_step(i, acc):
        return jnp.maximum(acc, vf[pl.ds(PROBS + i * LANES, LANES)])

    accg = lax.fori_loop(0, LANES, _mx2_step,
                         jnp.full((LANES,), -3.4e38, jnp.float32), unroll=4)
    shift = _bmax(accg).astype(jnp.int32) - 3

    hX.wait()
    hY.wait()

    zi = jnp.zeros((LANES,), jnp.int32)

    def _red_row(r, carry):
        accs = list(carry)
        rowv = jnp.broadcast_to(half * 32 + r, (LANES,))
        for jv in range(4):
            cid = vi[pl.ds(X0 + r * 64 + jv * 16, LANES)] + shift
            pkv = iota + (jv * 16 + (1 << 18))
            for c in range(NCELL):
                msk = cid == c
                accs[c] = accs[c] + jnp.where(msk, pkv, zi)
                accs[4 + c] = accs[4 + c] + jnp.where(msk, rowv, zi)
        return tuple(accs)

    accs = lax.fori_loop(0, 32, _red_row, tuple(zi for _ in range(8)),
                         unroll=2)
    pv = zf
    for c in range(NCELL):
        pk = _bsum(accs[c])
        cntc = (pk >> 18).astype(jnp.float32)
        sjc = (pk & ((1 << 18) - 1)).astype(jnp.float32)
        sic = _bsum(accs[4 + c]).astype(jnp.float32)
        pv = jnp.where(iota == c, cntc, pv)
        pv = jnp.where(iota == 4 + c, sic, pv)
        pv = jnp.where(iota == 8 + c, sjc, pv)
    vf[pl.ds(STAGE, LANES)] = pv
    pltpu.sync_copy(vf.at[pl.ds(STAGE, LANES)],
                    sh.at[pl.ds(SH_P + sub * LANES, LANES)])
    plsc.subcore_barrier()
    pltpu.sync_copy(sh.at[pl.ds(SH_P + partner * LANES, LANES)],
                    vf.at[pl.ds(PART, LANES)])
    tot = vf[pl.ds(STAGE, LANES)] + vf[pl.ds(PART, LANES)]

    qs = iota >> 2
    qd = iota & 3
    cnt_s = _shuf(tot, qs)
    cnt_d = _shuf(tot, qd)
    ch_s = _shuf(tot, qs + 4) / cnt_s
    ch_d = _shuf(tot, qd + 4) / cnt_d
    cw_s = _shuf(tot, qs + 8) / cnt_s
    cw_d = _shuf(tot, qd + 8) / cnt_d
    dh = ch_s - ch_d
    dw = cw_s - cw_d
    d2 = dh * dh + dw * dw
    cntm = jnp.where(iota < 4, tot, jnp.full((LANES,), -1.0, jnp.float32))
    ism = cntm == _bmax(cntm)
    score = jnp.where(ism, 16 - iota, jnp.zeros((LANES,), jnp.int32))
    med = 16 - _bmax(score)
    ok = ((d2 <= DIST2) & (cnt_s > 0.0) & (cnt_d > 0.0)
          & (qs != med) & (qd != med))
    vf[pl.ds(ADJ, LANES)] = jnp.where(ok, 1.0, 0.0)

    hW.wait()
    base_k = half * 10

    def _mlp_class(k, carry):
        kk = base_k + k
        t = kk & 3
        p = kk >> 2
        wrow_t = vf[pl.ds(WENC + (1 + t) * EMB, EMB)]
        wrow_0 = vf[pl.ds(WENC, EMB)]
        adjv = vf[pl.ds(ADJ, LANES)]
        hs = [jnp.maximum(wrow_t + wrow_0 * jnp.where(p == c, 1.0, 0.0), 0.0)
              for c in range(NCELL)]
        for l in range(NUM_LAYERS):
            aggs = []
            for d in range(NCELL):
                agg = zf
                for s in range(NCELL):
                    agg = agg + adjv[s * 4 + d] * hs[s]
                aggs.append(agg)
            new_hs = []
            for d in range(NCELL):
                acc = zf
                for e in range(EMB):
                    acc = acc + hs[d][e] * vf[pl.ds(WSELF + l * 256 + e * EMB, EMB)]
                    acc = acc + aggs[d][e] * vf[pl.ds(WNBR + l * 256 + e * EMB, EMB)]
                new_hs.append(jnp.maximum(acc, 0.0))
            hs = new_hs
        wd = vf[pl.ds(WDEC, EMB)]
        tv = zf
        for c in range(NCELL):
            tv = jnp.where(iota == c, _bsum(hs[c] * wd), tv)
        vf[pl.ds(TABLE + kk * LANES, LANES)] = tv
        return carry

    lax.fori_loop(0, 10, _mlp_class, 0)
    pltpu.sync_copy(vf.at[pl.ds(TABLE + half * 160, 160)],
                    sh.at[pl.ds(SH_T + sub * 160, 160)])
    plsc.subcore_barrier()
    pltpu.sync_copy(sh.at[pl.ds(SH_T + partner * 160, 160)],
                    vf.at[pl.ds(TABLE + (1 - half) * 160, 160)])

    T0 = [zf, zf, zf, zf]
    T1 = [zf, zf, zf, zf]
    for k in range(16):
        row_k = vf[pl.ds(TABLE + k * LANES, LANES)]
        for c in range(NCELL):
            T0[c] = jnp.where(iota == k, row_k[c], T0[c])
    for k in range(4):
        row_k = vf[pl.ds(TABLE + (16 + k) * LANES, LANES)]
        for c in range(NCELL):
            T1[c] = jnp.where(iota == k, row_k[c], T1[c])
    mxg = _bmax(jnp.maximum(
        jnp.maximum(jnp.maximum(T0[0], T0[1]), jnp.maximum(T0[2], T0[3])),
        jnp.maximum(jnp.maximum(T1[0], T1[1]), jnp.maximum(T1[2], T1[3]))))
    E0 = [jnp.exp(v - mxg) for v in T0]
    E1 = [jnp.exp(v - mxg) for v in T1]

    def _row(r, carry):
        rbase = r * 64
        Ls = []
        for jv in range(4):
            cid = vi[pl.ds(X0 + rbase + jv * 16, LANES)] + shift
            t = vi[pl.ds(X1 + rbase + jv * 16, LANES)]
            valid = cid >= 0
            k0 = jnp.where(valid, cid * 4 + t, 0)
            Ls.append([jnp.where(valid, _shuf(E0[c], k0), _shuf(E1[c], t))
                       for c in range(NCELL)])
        for c in range(NCELL):
            l0, l1, l2, l3 = (Ls[0][c], Ls[1][c], Ls[2][c], Ls[3][c])
            inv = 1.0 / _bsum((l0 + l1) + (l2 + l3))
            for jv in range(4):
                vf[pl.ds(PROBS + c * HPIX + rbase + jv * 16, LANES)] = (
                    Ls[jv][c] * inv + EPS)
        for jv in range(4):
            a0, a1, a2, a3 = (Ls[jv][0], Ls[jv][1], Ls[jv][2], Ls[jv][3])
            pm = jnp.maximum(jnp.maximum(a0, a1), jnp.maximum(a2, a3))
            arg = jnp.full((LANES,), 3, jnp.int32)
            arg = jnp.where(a2 == pm, 2, arg)
            arg = jnp.where(a1 == pm, 1, arg)
            arg = jnp.where(a0 == pm, 0, arg)
            vi[pl.ds(PRED + rbase + jv * 16, LANES)] = arg
        return carry

    lax.fori_loop(0, 32, _row, 0, unroll=2)

    hs_out = []
    for c in range(NCELL):
        hs_out.append(pltpu.async_copy(
            vf.at[pl.ds(PROBS + c * HPIX, HPIX)],
            probs_hbm.at[pl.ds(((b * NCELL + c) * H + half * 32) * W, HPIX)],
            sem_out))
    hs_out.append(pltpu.async_copy(
        vi.at[pl.ds(PRED, HPIX)],
        pred_hbm.at[pl.ds((b * H + half * 32) * W, HPIX)], sem_out))
    for h in hs_out:
        h.wait()


@jax.jit
def kernel(x, x_true, W_enc, W_self, W_nbr, W_dec):
    xflat = x.reshape(-1)
    xtflat = x_true.reshape(-1)
    wcat = jnp.concatenate([W_enc.reshape(-1), W_self.reshape(-1),
                            W_nbr.reshape(-1), W_dec.reshape(-1)])

    mesh = plsc.VectorSubcoreMesh(core_axis_name="c", subcore_axis_name="s")
    run = functools.partial(
        pl.kernel,
        mesh=mesh,
        out_type=[
            jax.ShapeDtypeStruct((B * NCELL * H * W,), jnp.float32),
            jax.ShapeDtypeStruct((B * H * W,), jnp.int32),
        ],
        scratch_types=[
            pltpu.VMEM((NI32,), jnp.int32),
            pltpu.VMEM((NF32,), jnp.float32),
            pltpu.VMEM_SHARED((NSH,), jnp.float32),
            pltpu.SemaphoreType.DMA,
            pltpu.SemaphoreType.DMA,
        ],
    )(_sc_body)

    probsf, predf = run(xflat, xtflat, wcat)
    return probsf.reshape(B, NCELL, H, W), predf.reshape(B, H, W)
